# baseline (device time: 18616 ns/iter reference)
import jax
import jax.numpy as jnp
from jax import lax
from jax.experimental import pallas as pl
from jax.experimental.pallas import tpu as pltpu

N_CHUNKS = 8


def kernel(x):
    m, n = x.shape
    q = m // N_CHUNKS
    half = N_CHUNKS // 2

    def body(x_ref, out_ref, recv_buf, send_sems, recv_sems):
        my_x = lax.axis_index("x")
        my_y = lax.axis_index("y")
        x_nbr = (1 - my_x, my_y)
        y_nbr = (my_x, 1 - my_y)

        barrier_sem = pltpu.get_barrier_semaphore()
        for nbr in (x_nbr, y_nbr):
            pl.semaphore_signal(
                barrier_sem, inc=1,
                device_id=nbr, device_id_type=pl.DeviceIdType.MESH,
            )
        pl.semaphore_wait(barrier_sem, 2)

        def chunk(ref, c):
            return ref.at[pl.ds(c * q, q), :]

        def mk_rdma(phase, c, nbr):
            src = chunk(x_ref if phase == 0 else out_ref, c)
            sem = N_CHUNKS * phase + c
            return pltpu.make_async_remote_copy(
                src_ref=src,
                dst_ref=recv_buf.at[phase, c],
                send_sem=send_sems.at[sem],
                recv_sem=recv_sems.at[sem],
                device_id=nbr,
                device_id_type=pl.DeviceIdType.MESH,
            )

        nbr0 = [x_nbr] * half + [y_nbr] * half
        nbr1 = [y_nbr] * half + [x_nbr] * half
        order = [c for pair in zip(range(half), range(half, N_CHUNKS))
                 for c in pair]

        p0 = {}
        for c in order:
            p0[c] = mk_rdma(0, c, nbr0[c])
            p0[c].start()
        p1 = {}
        for c in order:
            p0[c].wait()
            chunk(out_ref, c)[...] = chunk(x_ref, c)[...] + recv_buf[0, c]
            p1[c] = mk_rdma(1, c, nbr1[c])
            p1[c].start()
        for c in order:
            p1[c].wait()
            chunk(out_ref, c)[...] += recv_buf[1, c]

    return pl.pallas_call(
        body,
        out_shape=jax.ShapeDtypeStruct((m, n), jnp.float32),
        in_specs=[pl.BlockSpec(memory_space=pltpu.VMEM)],
        out_specs=pl.BlockSpec(memory_space=pltpu.VMEM),
        scratch_shapes=[
            pltpu.VMEM((2, N_CHUNKS, q, n), jnp.float32),
            pltpu.SemaphoreType.DMA((2 * N_CHUNKS,)),
            pltpu.SemaphoreType.DMA((2 * N_CHUNKS,)),
        ],
        compiler_params=pltpu.CompilerParams(collective_id=0),
    )(x)
